# baseline (speedup 1.0000x reference)
import functools

import jax
import jax.numpy as jnp
import numpy as np
from jax import lax
from jax.experimental import pallas as pl
from jax.experimental.pallas import tpu as pltpu
from jax.experimental.pallas import tpu_sc as plsc

F32 = jnp.float32
NEG_INF = float("-inf")


def _lanefold(x, op):
    while x.shape[1] > 128:
        h = x.shape[1] // 2
        x = op(x[:, :h], x[:, h:])
    return x


def _fps_kernel(p_ref, out_ref, *, n_samples):
    p = p_ref[...]
    _, rows, cols = p.shape
    p0, p1, p2 = p[0], p[1], p[2]
    flatf = (lax.broadcasted_iota(jnp.int32, (rows, cols), 0) * cols
             + lax.broadcasted_iota(jnp.int32, (rows, cols), 1)).astype(F32)
    bigf = jnp.float32(1e9)
    fmax = jnp.maximum
    fmin = jnp.minimum
    fadd = lambda a, b: a + b

    def dist_to_sel(sel):
        s0 = _lanefold(jnp.where(sel, p0, 0.0), fadd)
        s1 = _lanefold(jnp.where(sel, p1, 0.0), fadd)
        s2 = _lanefold(jnp.where(sel, p2, 0.0), fadd)
        st = jnp.concatenate([s0, s1, s2], axis=0)
        q = jnp.sum(st, axis=1, keepdims=True).reshape(3, rows, 1)
        q = jnp.sum(q, axis=1, keepdims=True)
        d0 = p0 - q[0]
        d1 = p1 - q[1]
        d2 = p2 - q[2]
        return d0 * d0 + d1 * d1 + d2 * d2

    out_ref[0] = 0
    min_d = dist_to_sel(flatf == 0.0)

    def step(i, md):
        m = jnp.max(_lanefold(md, fmax), axis=(0, 1), keepdims=True)
        idxf = jnp.min(_lanefold(jnp.where(md == m, flatf, bigf), fmin),
                       axis=(0, 1), keepdims=True)
        out_ref[i] = idxf[0, 0].astype(jnp.int32)
        return fmin(md, dist_to_sel(flatf == idxf))

    lax.fori_loop(1, n_samples, step, min_d, unroll=2)


def _fps(pos, n_samples):
    n = pos.shape[0]
    p = pos.T.reshape(3, 8, n // 8)
    return pl.pallas_call(
        functools.partial(_fps_kernel, n_samples=n_samples),
        out_shape=jax.ShapeDtypeStruct((n_samples,), jnp.int32),
        out_specs=pl.BlockSpec(memory_space=pltpu.SMEM),
    )(p)


def _q_kernel(x_ref, p_ref, wx_ref, wp_ref, b_ref, o_ref):
    o_ref[...] = (
        jnp.dot(x_ref[...], wx_ref[...],
                preferred_element_type=F32)
        + jnp.dot(p_ref[...], wp_ref[...],
                  preferred_element_type=F32)
        + b_ref[...]
    )


def _qmat(x, pos, wx, wp, b):
    n, f = x.shape
    h = wx.shape[1]
    bn = min(n, 1024)
    grid = n // bn
    return pl.pallas_call(
        _q_kernel,
        grid=(grid,),
        in_specs=[
            pl.BlockSpec((bn, f), lambda i: (i, 0)),
            pl.BlockSpec((bn, 3), lambda i: (i, 0)),
            pl.BlockSpec((f, h), lambda i: (0, 0)),
            pl.BlockSpec((3, h), lambda i: (0, 0)),
            pl.BlockSpec((1, h), lambda i: (0, 0)),
        ],
        out_specs=pl.BlockSpec((bn, h), lambda i: (i, 0)),
        out_shape=jax.ShapeDtypeStruct((n, h), F32),
    )(x, pos, wx, wp, b.reshape(1, h))


def _sa_edge_kernel(qg_ref, cen_ref, mask_ref, wp_ref, w2_ref, b2_ref,
                    w3_ref, b3_ref, o_ref, *, bc, nb):
    h = qg_ref.shape[1]
    c1 = jnp.dot(cen_ref[...], wp_ref[...], preferred_element_type=F32)
    c1e = jnp.broadcast_to(c1[:, None, :], (bc, nb, h)).reshape(bc * nb, h)
    h1 = jnp.maximum(qg_ref[...] - c1e, 0.0)
    h2 = jnp.maximum(
        jnp.dot(h1, w2_ref[...], preferred_element_type=F32) + b2_ref[...],
        0.0)
    msg = jnp.dot(h2, w3_ref[...], preferred_element_type=F32) + b3_ref[...]
    oc = msg.shape[1]
    msg = msg.reshape(bc, nb, oc)
    msg = jnp.where(mask_ref[...][:, :, None] > 0, msg, NEG_INF)
    o_ref[...] = jnp.max(msg, axis=1)


def _sa_edge(qg, centers, mask, wp, w2, b2, w3, b3, nb):
    nc = centers.shape[0]
    h = qg.shape[1]
    oc = w3.shape[1]
    bc = 8
    grid = nc // bc
    return pl.pallas_call(
        functools.partial(_sa_edge_kernel, bc=bc, nb=nb),
        grid=(grid,),
        in_specs=[
            pl.BlockSpec((bc * nb, h), lambda i: (i, 0)),
            pl.BlockSpec((bc, 3), lambda i: (i, 0)),
            pl.BlockSpec((bc, nb), lambda i: (i, 0)),
            pl.BlockSpec((3, h), lambda i: (0, 0)),
            pl.BlockSpec((h, h), lambda i: (0, 0)),
            pl.BlockSpec((1, h), lambda i: (0, 0)),
            pl.BlockSpec((h, oc), lambda i: (0, 0)),
            pl.BlockSpec((1, oc), lambda i: (0, 0)),
        ],
        out_specs=pl.BlockSpec((bc, oc), lambda i: (i, 0)),
        out_shape=jax.ShapeDtypeStruct((nc, oc), F32),
    )(qg, centers, mask, wp, w2, b2.reshape(1, h), w3, b3.reshape(1, oc))


def _select_kernel(cen_ref, pst_ref, m_ref, cnt_ref, *, r2, r2bits, nb):
    cen = cen_ref[...]
    pst = pst_ref[...]
    bc = cen.shape[0]
    aa = jnp.sum(cen * cen, axis=1, keepdims=True)
    bb = jnp.sum(pst * pst, axis=0, keepdims=True)
    d2 = jnp.maximum(
        aa + bb - 2.0 * jnp.dot(cen, pst, preferred_element_type=F32), 0.0)
    within = d2 <= r2
    d2b = lax.bitcast_convert_type(d2, jnp.int32)
    d2m = jnp.where(within, d2b, jnp.int32(2 ** 31 - 1))
    lo = jnp.full((bc, 1), -1, jnp.int32)
    hi = jnp.full((bc, 1), r2bits, jnp.int32)
    for _ in range(31):
        mid = (lo + hi) >> 1
        cnt = jnp.sum((d2m <= mid).astype(jnp.int32), axis=1, keepdims=True)
        ge = cnt >= nb
        hi = jnp.where(ge, mid, hi)
        lo = jnp.where(ge, lo, mid)
    m = within & (d2b <= hi)
    m_ref[...] = m.astype(jnp.int32)
    cnt_ref[...] = jnp.minimum(
        jnp.sum(m.astype(jnp.int32), axis=1, keepdims=True), nb)


def _select(centers, pst, r, nb):
    nc = centers.shape[0]
    n = pst.shape[1]
    r2 = float(np.float32(r * r))
    r2bits = int(np.float32(r * r).view(np.int32))
    bc = min(nc, 256)
    grid = nc // bc
    return pl.pallas_call(
        functools.partial(_select_kernel, r2=r2, r2bits=r2bits, nb=nb),
        grid=(grid,),
        in_specs=[
            pl.BlockSpec((bc, 3), lambda i: (i, 0)),
            pl.BlockSpec((3, n), lambda i: (0, 0)),
        ],
        out_specs=[
            pl.BlockSpec((bc, n), lambda i: (i, 0)),
            pl.BlockSpec((bc, 1), lambda i: (i, 0)),
        ],
        out_shape=[
            jax.ShapeDtypeStruct((nc, n), jnp.int32),
            jax.ShapeDtypeStruct((nc, 1), jnp.int32),
        ],
    )(centers, pst)


def _sc_compact(m, nb):
    nc, n = m.shape
    nw = 32
    rows_w = nc // nw
    mesh = plsc.VectorSubcoreMesh(core_axis_name="c", subcore_axis_name="s")

    @functools.partial(
        pl.kernel,
        mesh=mesh,
        compiler_params=pltpu.CompilerParams(needs_layout_passes=False),
        out_type=jax.ShapeDtypeStruct((nc, nb), jnp.int32),
        scratch_types=[
            pltpu.VMEM((n,), jnp.int32),
            pltpu.VMEM((rows_w, nb), jnp.int32),
        ],
    )
    def k(m_hbm, out_hbm, mrow_v, out_v):
        wid = lax.axis_index("s") * 2 + lax.axis_index("c")
        base = wid * rows_w
        zero = jnp.zeros((16,), jnp.int32)
        iota = lax.iota(jnp.int32, 16)

        def zrow(i, _):
            def zcol(j, __):
                out_v[i, pl.ds(j * 16, 16)] = zero
                return 0
            return lax.fori_loop(0, nb // 16, zcol, 0)

        lax.fori_loop(0, rows_w, zrow, 0)

        def row_loop(rr, _):
            pltpu.sync_copy(m_hbm.at[base + rr], mrow_v)
            rowvec = jnp.full((16,), rr, jnp.int32)

            def chunk(ci, cur):
                mv = mrow_v[pl.ds(ci * 16, 16)] != 0
                mi = jnp.where(mv, 1, 0).astype(jnp.int32)
                pos = plsc.cumsum(mi)
                tgt = jnp.minimum(cur + pos - 1, nb - 1)
                iv = ci * 16 + iota
                plsc.store_scatter(out_v, [rowvec, tgt], iv, mask=mv)
                return cur + jnp.sum(mi)

            lax.fori_loop(0, n // 16, chunk, jnp.int32(0))
            return 0

        lax.fori_loop(0, rows_w, row_loop, 0)
        pltpu.sync_copy(out_v, out_hbm.at[pl.ds(base, rows_w)])

    return k(m)


def _sa_module(x, pos, pst, centers, params, r, nb=64):
    (w1, b1), (w2, b2), (w3, b3) = params
    f = x.shape[1]
    wx, wp = w1[:f], w1[f:]
    q = _qmat(x, pos, wx, wp, b1)
    m, cnt = _select(centers, pst, r, nb)
    idx = _sc_compact(m, nb)
    mask = (jnp.arange(nb, dtype=jnp.int32)[None, :] < cnt).astype(F32)
    qg = jnp.take(q, idx.reshape(-1), axis=0)
    return _sa_edge(qg, centers, mask, wp, w2, b2, w3, b3, nb)


def _fp_kernel(pd_ref, pst_ref, xs_ref, skip_ref, wi_ref, ws_ref, b1_ref,
               w2_ref, b2_ref, *rest, k, three_layers):
    if three_layers:
        w3_ref, b3_ref, o_ref = rest
    else:
        (o_ref,) = rest
    pd = pd_ref[...]
    pst = pst_ref[...]
    bd = pd.shape[0]
    ns = pst.shape[1]
    aa = jnp.sum(pd * pd, axis=1, keepdims=True)
    bb = jnp.sum(pst * pst, axis=0, keepdims=True)
    d2 = jnp.maximum(
        aa + bb - 2.0 * jnp.dot(pd, pst, preferred_element_type=F32), 0.0)

    iot = lax.broadcasted_iota(jnp.int32, (bd, ns), 1)
    d2w = d2
    wa = jnp.zeros((bd, ns), F32)
    sumw = jnp.zeros((bd, 1), F32)
    for _ in range(k):
        m = jnp.min(d2w, axis=1, keepdims=True)
        sel = jnp.min(jnp.where(d2w == m, iot, ns), axis=1, keepdims=True)
        hit = iot == sel
        sx = jnp.sum(jnp.where(hit, pst[0:1, :], 0.0), axis=1, keepdims=True)
        sy = jnp.sum(jnp.where(hit, pst[1:2, :], 0.0), axis=1, keepdims=True)
        sz = jnp.sum(jnp.where(hit, pst[2:3, :], 0.0), axis=1, keepdims=True)
        dx = pd[:, 0:1] - sx
        dy = pd[:, 1:2] - sy
        dz = pd[:, 2:3] - sz
        d2g = dx * dx + dy * dy + dz * dz
        w = 1.0 / jnp.maximum(d2g, 1e-16)
        wa = wa + jnp.where(hit, w, 0.0)
        sumw = sumw + w
        d2w = jnp.where(hit, jnp.inf, d2w)

    interp = (jnp.dot(wa, xs_ref[...],
                      preferred_element_type=F32) / sumw)
    h = jnp.maximum(
        jnp.dot(interp, wi_ref[...], preferred_element_type=F32)
        + jnp.dot(skip_ref[...], ws_ref[...],
                  preferred_element_type=F32)
        + b1_ref[...], 0.0)
    out = jnp.dot(h, w2_ref[...], preferred_element_type=F32) + b2_ref[...]
    if three_layers:
        out = jnp.maximum(out, 0.0)
        out = jnp.dot(out, w3_ref[...], preferred_element_type=F32) + b3_ref[...]
    o_ref[...] = out


def _fp_module(x_src, pos_src, pos_dst, skip, params, k):
    nd = pos_dst.shape[0]
    ns = pos_src.shape[0]
    c = x_src.shape[1]
    s = skip.shape[1]
    three_layers = len(params) == 3
    (w1, b1) = params[0]
    wi, ws = w1[:c], w1[c:]
    (w2, b2) = params[1]
    h1 = w2.shape[0]
    oc = w2.shape[1]
    bd = min(nd, 512)
    grid = nd // bd
    pst = pos_src.T.reshape(3, ns)

    in_specs = [
        pl.BlockSpec((bd, 3), lambda i: (i, 0)),
        pl.BlockSpec((3, ns), lambda i: (0, 0)),
        pl.BlockSpec((ns, c), lambda i: (0, 0)),
        pl.BlockSpec((bd, s), lambda i: (i, 0)),
        pl.BlockSpec((c, h1), lambda i: (0, 0)),
        pl.BlockSpec((s, h1), lambda i: (0, 0)),
        pl.BlockSpec((1, h1), lambda i: (0, 0)),
        pl.BlockSpec((h1, oc), lambda i: (0, 0)),
        pl.BlockSpec((1, oc), lambda i: (0, 0)),
    ]
    args = [pos_dst, pst, x_src, skip, wi, ws, b1.reshape(1, h1), w2,
            b2.reshape(1, oc)]
    out_c = oc
    if three_layers:
        (w3, b3) = params[2]
        out_c = w3.shape[1]
        in_specs += [
            pl.BlockSpec((oc, out_c), lambda i: (0, 0)),
            pl.BlockSpec((1, out_c), lambda i: (0, 0)),
        ]
        args += [w3, b3.reshape(1, out_c)]

    return pl.pallas_call(
        functools.partial(_fp_kernel, k=k, three_layers=three_layers),
        grid=(grid,),
        in_specs=in_specs,
        out_specs=pl.BlockSpec((bd, out_c), lambda i: (i, 0)),
        out_shape=jax.ShapeDtypeStruct((nd, out_c), F32),
    )(*args)


def kernel(x, pos, batch, params):
    del batch
    samp1 = _fps(pos, pos.shape[0] // 4)
    pos1 = jnp.take(pos, samp1, axis=0)
    x1 = _sa_module(x, pos, pos.T, pos1, params['sa1'], 0.2)

    samp2 = _fps(pos1, pos1.shape[0] // 4)
    pos2 = jnp.take(pos1, samp2, axis=0)
    x2 = _sa_module(x1, pos1, pos1.T, pos2, params['sa2'], 0.4)

    samp3 = _fps(pos2, pos2.shape[0] // 4)
    pos3 = jnp.take(pos2, samp3, axis=0)
    x3 = _sa_module(x2, pos2, pos2.T, pos3, params['sa3'], 0.8)

    f3 = _fp_module(x3, pos3, pos2, x2, params['fp3'], 1)
    f2 = _fp_module(f3, pos2, pos1, x1, params['fp2'], 3)
    f1 = _fp_module(f2, pos1, pos, x, params['fp1'], 3)
    return f1


# speedup vs baseline: 1.0480x; 1.0480x over previous
import functools

import jax
import jax.numpy as jnp
import numpy as np
from jax import lax
from jax.experimental import pallas as pl
from jax.experimental.pallas import tpu as pltpu
from jax.experimental.pallas import tpu_sc as plsc

F32 = jnp.float32
NEG_INF = float("-inf")


def _lanefold(x, op):
    while x.shape[1] > 128:
        h = x.shape[1] // 2
        x = op(x[:, :h], x[:, h:])
    return x


def _fps_kernel(p_ref, out_ref, *, n_samples):
    p = p_ref[...]
    _, rows, cols = p.shape
    p0, p1, p2 = p[0], p[1], p[2]
    flatf = (lax.broadcasted_iota(jnp.int32, (rows, cols), 0) * cols
             + lax.broadcasted_iota(jnp.int32, (rows, cols), 1)).astype(F32)
    bigf = jnp.float32(1e9)
    fmax = jnp.maximum
    fmin = jnp.minimum
    fadd = lambda a, b: a + b

    def dist_to_sel(sel):
        s0 = _lanefold(jnp.where(sel, p0, 0.0), fadd)
        s1 = _lanefold(jnp.where(sel, p1, 0.0), fadd)
        s2 = _lanefold(jnp.where(sel, p2, 0.0), fadd)
        st = jnp.concatenate([s0, s1, s2], axis=0)
        q = jnp.sum(st, axis=1, keepdims=True).reshape(3, rows, 1)
        q = jnp.sum(q, axis=1, keepdims=True)
        d0 = p0 - q[0]
        d1 = p1 - q[1]
        d2 = p2 - q[2]
        return d0 * d0 + d1 * d1 + d2 * d2

    out_ref[0] = 0
    min_d = dist_to_sel(flatf == 0.0)

    def step(i, md):
        m = jnp.max(_lanefold(md, fmax), axis=(0, 1), keepdims=True)
        idxf = jnp.min(_lanefold(jnp.where(md == m, flatf, bigf), fmin),
                       axis=(0, 1), keepdims=True)
        out_ref[i] = idxf[0, 0].astype(jnp.int32)
        return fmin(md, dist_to_sel(flatf == idxf))

    lax.fori_loop(1, n_samples, step, min_d, unroll=2)


def _fps(pos, n_samples):
    n = pos.shape[0]
    p = pos.T.reshape(3, 8, n // 8)
    return pl.pallas_call(
        functools.partial(_fps_kernel, n_samples=n_samples),
        out_shape=jax.ShapeDtypeStruct((n_samples,), jnp.int32),
        out_specs=pl.BlockSpec(memory_space=pltpu.SMEM),
    )(p)


def _q_kernel(x_ref, p_ref, wx_ref, wp_ref, b_ref, o_ref):
    o_ref[...] = (
        jnp.dot(x_ref[...], wx_ref[...],
                preferred_element_type=F32)
        + jnp.dot(p_ref[...], wp_ref[...],
                  preferred_element_type=F32)
        + b_ref[...]
    )


def _qmat(x, pos, wx, wp, b):
    n, f = x.shape
    h = wx.shape[1]
    bn = min(n, 1024)
    grid = n // bn
    return pl.pallas_call(
        _q_kernel,
        grid=(grid,),
        in_specs=[
            pl.BlockSpec((bn, f), lambda i: (i, 0)),
            pl.BlockSpec((bn, 3), lambda i: (i, 0)),
            pl.BlockSpec((f, h), lambda i: (0, 0)),
            pl.BlockSpec((3, h), lambda i: (0, 0)),
            pl.BlockSpec((1, h), lambda i: (0, 0)),
        ],
        out_specs=pl.BlockSpec((bn, h), lambda i: (i, 0)),
        out_shape=jax.ShapeDtypeStruct((n, h), F32),
    )(x, pos, wx, wp, b.reshape(1, h))


def _sa_edge_kernel(qg_ref, cen_ref, mask_ref, wp_ref, w2_ref, b2_ref,
                    w3_ref, b3_ref, o_ref, *, bc, nb):
    h = qg_ref.shape[1]
    c1 = jnp.dot(cen_ref[...], wp_ref[...], preferred_element_type=F32)
    c1e = jnp.broadcast_to(c1[:, None, :], (bc, nb, h)).reshape(bc * nb, h)
    h1 = jnp.maximum(qg_ref[...] - c1e, 0.0)
    h2 = jnp.maximum(
        jnp.dot(h1, w2_ref[...], preferred_element_type=F32) + b2_ref[...],
        0.0)
    msg = jnp.dot(h2, w3_ref[...], preferred_element_type=F32) + b3_ref[...]
    oc = msg.shape[1]
    msg = msg.reshape(bc, nb, oc)
    msg = jnp.where(mask_ref[...][:, :, None] > 0, msg, NEG_INF)
    o_ref[...] = jnp.max(msg, axis=1)


def _sa_edge(qg, centers, mask, wp, w2, b2, w3, b3, nb):
    nc = centers.shape[0]
    h = qg.shape[1]
    oc = w3.shape[1]
    bc = 16
    grid = nc // bc
    return pl.pallas_call(
        functools.partial(_sa_edge_kernel, bc=bc, nb=nb),
        grid=(grid,),
        in_specs=[
            pl.BlockSpec((bc * nb, h), lambda i: (i, 0)),
            pl.BlockSpec((bc, 3), lambda i: (i, 0)),
            pl.BlockSpec((bc, nb), lambda i: (i, 0)),
            pl.BlockSpec((3, h), lambda i: (0, 0)),
            pl.BlockSpec((h, h), lambda i: (0, 0)),
            pl.BlockSpec((1, h), lambda i: (0, 0)),
            pl.BlockSpec((h, oc), lambda i: (0, 0)),
            pl.BlockSpec((1, oc), lambda i: (0, 0)),
        ],
        out_specs=pl.BlockSpec((bc, oc), lambda i: (i, 0)),
        out_shape=jax.ShapeDtypeStruct((nc, oc), F32),
    )(qg, centers, mask, wp, w2, b2.reshape(1, h), w3, b3.reshape(1, oc))


def _select_kernel(cen_ref, pst_ref, m_ref, cnt_ref, *, r2, r2bits, nb):
    cen = cen_ref[...]
    pst = pst_ref[...]
    bc = cen.shape[0]
    aa = jnp.sum(cen * cen, axis=1, keepdims=True)
    bb = jnp.sum(pst * pst, axis=0, keepdims=True)
    d2 = jnp.maximum(
        aa + bb - 2.0 * jnp.dot(cen, pst, preferred_element_type=F32), 0.0)
    within = d2 <= r2
    d2b = lax.bitcast_convert_type(d2, jnp.int32)
    d2m = jnp.where(within, d2b, jnp.int32(2 ** 31 - 1))
    lo = jnp.full((bc, 1), -1, jnp.int32)
    hi = jnp.full((bc, 1), r2bits, jnp.int32)
    for _ in range(31):
        mid = (lo + hi) >> 1
        cnt = jnp.sum((d2m <= mid).astype(jnp.int32), axis=1, keepdims=True)
        ge = cnt >= nb
        hi = jnp.where(ge, mid, hi)
        lo = jnp.where(ge, lo, mid)
    m = within & (d2b <= hi)
    m_ref[...] = m.astype(jnp.int32)
    cnt_ref[...] = jnp.minimum(
        jnp.sum(m.astype(jnp.int32), axis=1, keepdims=True), nb)


def _select(centers, pst, r, nb):
    nc = centers.shape[0]
    n = pst.shape[1]
    r2 = float(np.float32(r * r))
    r2bits = int(np.float32(r * r).view(np.int32))
    bc = min(nc, 256)
    grid = nc // bc
    return pl.pallas_call(
        functools.partial(_select_kernel, r2=r2, r2bits=r2bits, nb=nb),
        grid=(grid,),
        in_specs=[
            pl.BlockSpec((bc, 3), lambda i: (i, 0)),
            pl.BlockSpec((3, n), lambda i: (0, 0)),
        ],
        out_specs=[
            pl.BlockSpec((bc, n), lambda i: (i, 0)),
            pl.BlockSpec((bc, 1), lambda i: (i, 0)),
        ],
        out_shape=[
            jax.ShapeDtypeStruct((nc, n), jnp.int32),
            jax.ShapeDtypeStruct((nc, 1), jnp.int32),
        ],
    )(centers, pst)


def _sc_compact(m, nb):
    nc, n = m.shape
    nw = 32
    rows_w = nc // nw
    mesh = plsc.VectorSubcoreMesh(core_axis_name="c", subcore_axis_name="s")

    @functools.partial(
        pl.kernel,
        mesh=mesh,
        compiler_params=pltpu.CompilerParams(needs_layout_passes=False),
        out_type=jax.ShapeDtypeStruct((nc, nb), jnp.int32),
        scratch_types=[
            pltpu.VMEM((n,), jnp.int32),
            pltpu.VMEM((rows_w, nb), jnp.int32),
        ],
    )
    def k(m_hbm, out_hbm, mrow_v, out_v):
        wid = lax.axis_index("s") * 2 + lax.axis_index("c")
        base = wid * rows_w
        zero = jnp.zeros((16,), jnp.int32)
        iota = lax.iota(jnp.int32, 16)

        def zrow(i, _):
            def zcol(j, __):
                out_v[i, pl.ds(j * 16, 16)] = zero
                return 0
            return lax.fori_loop(0, nb // 16, zcol, 0)

        lax.fori_loop(0, rows_w, zrow, 0)

        def row_loop(rr, _):
            pltpu.sync_copy(m_hbm.at[base + rr], mrow_v)
            rowvec = jnp.full((16,), rr, jnp.int32)

            def chunk(ci, cur):
                mv = mrow_v[pl.ds(ci * 16, 16)] != 0
                mi = jnp.where(mv, 1, 0).astype(jnp.int32)
                pos = plsc.cumsum(mi)
                tgt = jnp.minimum(cur + pos - 1, nb - 1)
                iv = ci * 16 + iota
                plsc.store_scatter(out_v, [rowvec, tgt], iv, mask=mv)
                return cur + jnp.sum(mi)

            lax.fori_loop(0, n // 16, chunk, jnp.int32(0))
            return 0

        lax.fori_loop(0, rows_w, row_loop, 0)
        pltpu.sync_copy(out_v, out_hbm.at[pl.ds(base, rows_w)])

    return k(m)


def _sa_module(x, pos, pst, centers, params, r, nb=64):
    (w1, b1), (w2, b2), (w3, b3) = params
    f = x.shape[1]
    wx, wp = w1[:f], w1[f:]
    q = _qmat(x, pos, wx, wp, b1)
    m, cnt = _select(centers, pst, r, nb)
    idx = _sc_compact(m, nb)
    mask = (jnp.arange(nb, dtype=jnp.int32)[None, :] < cnt).astype(F32)
    qg = jnp.take(q, idx.reshape(-1), axis=0)
    return _sa_edge(qg, centers, mask, wp, w2, b2, w3, b3, nb)


def _fp_kernel(pd_ref, pst_ref, xs_ref, skip_ref, wi_ref, ws_ref, b1_ref,
               w2_ref, b2_ref, *rest, k, three_layers):
    if three_layers:
        w3_ref, b3_ref, o_ref = rest
    else:
        (o_ref,) = rest
    pd = pd_ref[...]
    pst = pst_ref[...]
    bd = pd.shape[0]
    ns = pst.shape[1]
    aa = jnp.sum(pd * pd, axis=1, keepdims=True)
    bb = jnp.sum(pst * pst, axis=0, keepdims=True)
    d2 = jnp.maximum(
        aa + bb - 2.0 * jnp.dot(pd, pst, preferred_element_type=F32), 0.0)

    iot = lax.broadcasted_iota(jnp.int32, (bd, ns), 1)
    d2w = d2
    wa = jnp.zeros((bd, ns), F32)
    sumw = jnp.zeros((bd, 1), F32)
    for _ in range(k):
        m = jnp.min(d2w, axis=1, keepdims=True)
        sel = jnp.min(jnp.where(d2w == m, iot, ns), axis=1, keepdims=True)
        hit = iot == sel
        sx = jnp.sum(jnp.where(hit, pst[0:1, :], 0.0), axis=1, keepdims=True)
        sy = jnp.sum(jnp.where(hit, pst[1:2, :], 0.0), axis=1, keepdims=True)
        sz = jnp.sum(jnp.where(hit, pst[2:3, :], 0.0), axis=1, keepdims=True)
        dx = pd[:, 0:1] - sx
        dy = pd[:, 1:2] - sy
        dz = pd[:, 2:3] - sz
        d2g = dx * dx + dy * dy + dz * dz
        w = 1.0 / jnp.maximum(d2g, 1e-16)
        wa = wa + jnp.where(hit, w, 0.0)
        sumw = sumw + w
        d2w = jnp.where(hit, jnp.inf, d2w)

    interp = (jnp.dot(wa, xs_ref[...],
                      preferred_element_type=F32) / sumw)
    h = jnp.maximum(
        jnp.dot(interp, wi_ref[...], preferred_element_type=F32)
        + jnp.dot(skip_ref[...], ws_ref[...],
                  preferred_element_type=F32)
        + b1_ref[...], 0.0)
    out = jnp.dot(h, w2_ref[...], preferred_element_type=F32) + b2_ref[...]
    if three_layers:
        out = jnp.maximum(out, 0.0)
        out = jnp.dot(out, w3_ref[...], preferred_element_type=F32) + b3_ref[...]
    o_ref[...] = out


def _fp_module(x_src, pos_src, pos_dst, skip, params, k):
    nd = pos_dst.shape[0]
    ns = pos_src.shape[0]
    c = x_src.shape[1]
    s = skip.shape[1]
    three_layers = len(params) == 3
    (w1, b1) = params[0]
    wi, ws = w1[:c], w1[c:]
    (w2, b2) = params[1]
    h1 = w2.shape[0]
    oc = w2.shape[1]
    bd = min(nd, 512)
    grid = nd // bd
    pst = pos_src.T.reshape(3, ns)

    in_specs = [
        pl.BlockSpec((bd, 3), lambda i: (i, 0)),
        pl.BlockSpec((3, ns), lambda i: (0, 0)),
        pl.BlockSpec((ns, c), lambda i: (0, 0)),
        pl.BlockSpec((bd, s), lambda i: (i, 0)),
        pl.BlockSpec((c, h1), lambda i: (0, 0)),
        pl.BlockSpec((s, h1), lambda i: (0, 0)),
        pl.BlockSpec((1, h1), lambda i: (0, 0)),
        pl.BlockSpec((h1, oc), lambda i: (0, 0)),
        pl.BlockSpec((1, oc), lambda i: (0, 0)),
    ]
    args = [pos_dst, pst, x_src, skip, wi, ws, b1.reshape(1, h1), w2,
            b2.reshape(1, oc)]
    out_c = oc
    if three_layers:
        (w3, b3) = params[2]
        out_c = w3.shape[1]
        in_specs += [
            pl.BlockSpec((oc, out_c), lambda i: (0, 0)),
            pl.BlockSpec((1, out_c), lambda i: (0, 0)),
        ]
        args += [w3, b3.reshape(1, out_c)]

    return pl.pallas_call(
        functools.partial(_fp_kernel, k=k, three_layers=three_layers),
        grid=(grid,),
        in_specs=in_specs,
        out_specs=pl.BlockSpec((bd, out_c), lambda i: (i, 0)),
        out_shape=jax.ShapeDtypeStruct((nd, out_c), F32),
    )(*args)


def kernel(x, pos, batch, params):
    del batch
    samp1 = _fps(pos, pos.shape[0] // 4)
    pos1 = jnp.take(pos, samp1, axis=0)
    x1 = _sa_module(x, pos, pos.T, pos1, params['sa1'], 0.2)

    samp2 = _fps(pos1, pos1.shape[0] // 4)
    pos2 = jnp.take(pos1, samp2, axis=0)
    x2 = _sa_module(x1, pos1, pos1.T, pos2, params['sa2'], 0.4)

    samp3 = _fps(pos2, pos2.shape[0] // 4)
    pos3 = jnp.take(pos2, samp3, axis=0)
    x3 = _sa_module(x2, pos2, pos2.T, pos3, params['sa3'], 0.8)

    f3 = _fp_module(x3, pos3, pos2, x2, params['fp3'], 1)
    f2 = _fp_module(f3, pos2, pos1, x1, params['fp2'], 3)
    f1 = _fp_module(f2, pos1, pos, x, params['fp1'], 3)
    return f1


# speedup vs baseline: 1.0713x; 1.0222x over previous
import functools

import jax
import jax.numpy as jnp
import numpy as np
from jax import lax
from jax.experimental import pallas as pl
from jax.experimental.pallas import tpu as pltpu
from jax.experimental.pallas import tpu_sc as plsc

F32 = jnp.float32
NEG_INF = float("-inf")


def _lanefold(x, op):
    while x.shape[1] > 128:
        h = x.shape[1] // 2
        x = op(x[:, :h], x[:, h:])
    return x


def _fps_kernel(p_ref, out_ref, *, n_samples):
    p = p_ref[...]
    _, rows, cols = p.shape
    p0, p1, p2 = p[0], p[1], p[2]
    flatf = (lax.broadcasted_iota(jnp.int32, (rows, cols), 0) * cols
             + lax.broadcasted_iota(jnp.int32, (rows, cols), 1)).astype(F32)
    bigf = jnp.float32(1e9)
    fmax = jnp.maximum
    fmin = jnp.minimum
    fadd = lambda a, b: a + b

    def dist_to_sel(sel):
        s0 = _lanefold(jnp.where(sel, p0, 0.0), fadd)
        s1 = _lanefold(jnp.where(sel, p1, 0.0), fadd)
        s2 = _lanefold(jnp.where(sel, p2, 0.0), fadd)
        st = jnp.concatenate([s0, s1, s2], axis=0)
        q = jnp.sum(st, axis=1, keepdims=True).reshape(3, rows, 1)
        q = jnp.sum(q, axis=1, keepdims=True)
        d0 = p0 - q[0]
        d1 = p1 - q[1]
        d2 = p2 - q[2]
        return d0 * d0 + d1 * d1 + d2 * d2

    out_ref[0] = 0
    min_d = dist_to_sel(flatf == 0.0)

    def step(i, md):
        m = jnp.max(_lanefold(md, fmax), axis=(0, 1), keepdims=True)
        idxf = jnp.min(_lanefold(jnp.where(md == m, flatf, bigf), fmin),
                       axis=(0, 1), keepdims=True)
        out_ref[i] = idxf[0, 0].astype(jnp.int32)
        return fmin(md, dist_to_sel(flatf == idxf))

    lax.fori_loop(1, n_samples, step, min_d, unroll=2)


def _fps(pos, n_samples):
    n = pos.shape[0]
    p = pos.T.reshape(3, 8, n // 8)
    return pl.pallas_call(
        functools.partial(_fps_kernel, n_samples=n_samples),
        out_shape=jax.ShapeDtypeStruct((n_samples,), jnp.int32),
        out_specs=pl.BlockSpec(memory_space=pltpu.SMEM),
    )(p)


def _q_kernel(x_ref, p_ref, wx_ref, wp_ref, b_ref, o_ref):
    o_ref[...] = (
        jnp.dot(x_ref[...], wx_ref[...],
                preferred_element_type=F32)
        + jnp.dot(p_ref[...], wp_ref[...],
                  preferred_element_type=F32)
        + b_ref[...]
    )


def _qmat(x, pos, wx, wp, b):
    n, f = x.shape
    h = wx.shape[1]
    bn = min(n, 1024)
    grid = n // bn
    return pl.pallas_call(
        _q_kernel,
        grid=(grid,),
        in_specs=[
            pl.BlockSpec((bn, f), lambda i: (i, 0)),
            pl.BlockSpec((bn, 3), lambda i: (i, 0)),
            pl.BlockSpec((f, h), lambda i: (0, 0)),
            pl.BlockSpec((3, h), lambda i: (0, 0)),
            pl.BlockSpec((1, h), lambda i: (0, 0)),
        ],
        out_specs=pl.BlockSpec((bn, h), lambda i: (i, 0)),
        out_shape=jax.ShapeDtypeStruct((n, h), F32),
    )(x, pos, wx, wp, b.reshape(1, h))


def _sa_edge_kernel(qg_ref, cen_ref, mask_ref, wp_ref, w2_ref, b2_ref,
                    w3_ref, b3_ref, o_ref, *, bc, nb):
    h = qg_ref.shape[1]
    c1 = jnp.dot(cen_ref[...], wp_ref[...], preferred_element_type=F32)
    c1e = jnp.broadcast_to(c1[:, None, :], (bc, nb, h)).reshape(bc * nb, h)
    h1 = jnp.maximum(qg_ref[...] - c1e, 0.0)
    h2 = jnp.maximum(
        jnp.dot(h1, w2_ref[...], preferred_element_type=F32) + b2_ref[...],
        0.0)
    msg = jnp.dot(h2, w3_ref[...], preferred_element_type=F32) + b3_ref[...]
    oc = msg.shape[1]
    msg = msg.reshape(bc, nb, oc)
    msg = jnp.where(mask_ref[...][:, :, None] > 0, msg, NEG_INF)
    o_ref[...] = jnp.max(msg, axis=1)


def _sa_edge(qg, centers, mask, wp, w2, b2, w3, b3, nb):
    nc = centers.shape[0]
    h = qg.shape[1]
    oc = w3.shape[1]
    bc = 32
    grid = nc // bc
    return pl.pallas_call(
        functools.partial(_sa_edge_kernel, bc=bc, nb=nb),
        grid=(grid,),
        in_specs=[
            pl.BlockSpec((bc * nb, h), lambda i: (i, 0)),
            pl.BlockSpec((bc, 3), lambda i: (i, 0)),
            pl.BlockSpec((bc, nb), lambda i: (i, 0)),
            pl.BlockSpec((3, h), lambda i: (0, 0)),
            pl.BlockSpec((h, h), lambda i: (0, 0)),
            pl.BlockSpec((1, h), lambda i: (0, 0)),
            pl.BlockSpec((h, oc), lambda i: (0, 0)),
            pl.BlockSpec((1, oc), lambda i: (0, 0)),
        ],
        out_specs=pl.BlockSpec((bc, oc), lambda i: (i, 0)),
        out_shape=jax.ShapeDtypeStruct((nc, oc), F32),
    )(qg, centers, mask, wp, w2, b2.reshape(1, h), w3, b3.reshape(1, oc))


def _select_kernel(cen_ref, pst_ref, m_ref, cnt_ref, *, r2, r2bits, nb):
    cen = cen_ref[...]
    pst = pst_ref[...]
    bc = cen.shape[0]
    aa = jnp.sum(cen * cen, axis=1, keepdims=True)
    bb = jnp.sum(pst * pst, axis=0, keepdims=True)
    d2 = jnp.maximum(
        aa + bb - 2.0 * jnp.dot(cen, pst, preferred_element_type=F32), 0.0)
    within = d2 <= r2
    d2b = lax.bitcast_convert_type(d2, jnp.int32)
    d2m = jnp.where(within, d2b, jnp.int32(2 ** 31 - 1))
    lo = jnp.full((bc, 1), -1, jnp.int32)
    hi = jnp.full((bc, 1), r2bits, jnp.int32)
    for _ in range(31):
        mid = (lo + hi) >> 1
        cnt = jnp.sum((d2m <= mid).astype(jnp.int32), axis=1, keepdims=True)
        ge = cnt >= nb
        hi = jnp.where(ge, mid, hi)
        lo = jnp.where(ge, lo, mid)
    m = within & (d2b <= hi)
    m_ref[...] = m.astype(jnp.int32)
    cnt_ref[...] = jnp.minimum(
        jnp.sum(m.astype(jnp.int32), axis=1, keepdims=True), nb)


def _select(centers, pst, r, nb):
    nc = centers.shape[0]
    n = pst.shape[1]
    r2 = float(np.float32(r * r))
    r2bits = int(np.float32(r * r).view(np.int32))
    bc = min(nc, 256)
    grid = nc // bc
    return pl.pallas_call(
        functools.partial(_select_kernel, r2=r2, r2bits=r2bits, nb=nb),
        grid=(grid,),
        in_specs=[
            pl.BlockSpec((bc, 3), lambda i: (i, 0)),
            pl.BlockSpec((3, n), lambda i: (0, 0)),
        ],
        out_specs=[
            pl.BlockSpec((bc, n), lambda i: (i, 0)),
            pl.BlockSpec((bc, 1), lambda i: (i, 0)),
        ],
        out_shape=[
            jax.ShapeDtypeStruct((nc, n), jnp.int32),
            jax.ShapeDtypeStruct((nc, 1), jnp.int32),
        ],
    )(centers, pst)


def _sc_compact(m, nb):
    nc, n = m.shape
    nw = 32
    rows_w = nc // nw
    mesh = plsc.VectorSubcoreMesh(core_axis_name="c", subcore_axis_name="s")

    @functools.partial(
        pl.kernel,
        mesh=mesh,
        compiler_params=pltpu.CompilerParams(needs_layout_passes=False),
        out_type=jax.ShapeDtypeStruct((nc, nb), jnp.int32),
        scratch_types=[
            pltpu.VMEM((n,), jnp.int32),
            pltpu.VMEM((rows_w, nb), jnp.int32),
        ],
    )
    def k(m_hbm, out_hbm, mrow_v, out_v):
        wid = lax.axis_index("s") * 2 + lax.axis_index("c")
        base = wid * rows_w
        zero = jnp.zeros((16,), jnp.int32)
        iota = lax.iota(jnp.int32, 16)

        def zrow(i, _):
            def zcol(j, __):
                out_v[i, pl.ds(j * 16, 16)] = zero
                return 0
            return lax.fori_loop(0, nb // 16, zcol, 0)

        lax.fori_loop(0, rows_w, zrow, 0)

        def row_loop(rr, _):
            pltpu.sync_copy(m_hbm.at[base + rr], mrow_v)
            rowvec = jnp.full((16,), rr, jnp.int32)

            def chunk(ci, cur):
                mv = mrow_v[pl.ds(ci * 16, 16)] != 0
                mi = jnp.where(mv, 1, 0).astype(jnp.int32)
                pos = plsc.cumsum(mi)
                tgt = jnp.minimum(cur + pos - 1, nb - 1)
                iv = ci * 16 + iota
                plsc.store_scatter(out_v, [rowvec, tgt], iv, mask=mv)
                return cur + jnp.sum(mi)

            lax.fori_loop(0, n // 16, chunk, jnp.int32(0))
            return 0

        lax.fori_loop(0, rows_w, row_loop, 0)
        pltpu.sync_copy(out_v, out_hbm.at[pl.ds(base, rows_w)])

    return k(m)


def _sa_module(x, pos, pst, centers, params, r, nb=64):
    (w1, b1), (w2, b2), (w3, b3) = params
    f = x.shape[1]
    wx, wp = w1[:f], w1[f:]
    q = _qmat(x, pos, wx, wp, b1)
    m, cnt = _select(centers, pst, r, nb)
    idx = _sc_compact(m, nb)
    mask = (jnp.arange(nb, dtype=jnp.int32)[None, :] < cnt).astype(F32)
    qg = jnp.take(q, idx.reshape(-1), axis=0)
    return _sa_edge(qg, centers, mask, wp, w2, b2, w3, b3, nb)


def _fp_kernel(pd_ref, pst_ref, xs_ref, skip_ref, wi_ref, ws_ref, b1_ref,
               w2_ref, b2_ref, *rest, k, three_layers):
    if three_layers:
        w3_ref, b3_ref, o_ref = rest
    else:
        (o_ref,) = rest
    pd = pd_ref[...]
    pst = pst_ref[...]
    bd = pd.shape[0]
    ns = pst.shape[1]
    aa = jnp.sum(pd * pd, axis=1, keepdims=True)
    bb = jnp.sum(pst * pst, axis=0, keepdims=True)
    d2 = jnp.maximum(
        aa + bb - 2.0 * jnp.dot(pd, pst, preferred_element_type=F32), 0.0)

    iot = lax.broadcasted_iota(jnp.int32, (bd, ns), 1)
    d2w = d2
    wa = jnp.zeros((bd, ns), F32)
    sumw = jnp.zeros((bd, 1), F32)
    for _ in range(k):
        m = jnp.min(d2w, axis=1, keepdims=True)
        sel = jnp.min(jnp.where(d2w == m, iot, ns), axis=1, keepdims=True)
        hit = iot == sel
        sx = jnp.sum(jnp.where(hit, pst[0:1, :], 0.0), axis=1, keepdims=True)
        sy = jnp.sum(jnp.where(hit, pst[1:2, :], 0.0), axis=1, keepdims=True)
        sz = jnp.sum(jnp.where(hit, pst[2:3, :], 0.0), axis=1, keepdims=True)
        dx = pd[:, 0:1] - sx
        dy = pd[:, 1:2] - sy
        dz = pd[:, 2:3] - sz
        d2g = dx * dx + dy * dy + dz * dz
        w = 1.0 / jnp.maximum(d2g, 1e-16)
        wa = wa + jnp.where(hit, w, 0.0)
        sumw = sumw + w
        d2w = jnp.where(hit, jnp.inf, d2w)

    interp = (jnp.dot(wa, xs_ref[...],
                      preferred_element_type=F32) / sumw)
    h = jnp.maximum(
        jnp.dot(interp, wi_ref[...], preferred_element_type=F32)
        + jnp.dot(skip_ref[...], ws_ref[...],
                  preferred_element_type=F32)
        + b1_ref[...], 0.0)
    out = jnp.dot(h, w2_ref[...], preferred_element_type=F32) + b2_ref[...]
    if three_layers:
        out = jnp.maximum(out, 0.0)
        out = jnp.dot(out, w3_ref[...], preferred_element_type=F32) + b3_ref[...]
    o_ref[...] = out


def _fp_module(x_src, pos_src, pos_dst, skip, params, k):
    nd = pos_dst.shape[0]
    ns = pos_src.shape[0]
    c = x_src.shape[1]
    s = skip.shape[1]
    three_layers = len(params) == 3
    (w1, b1) = params[0]
    wi, ws = w1[:c], w1[c:]
    (w2, b2) = params[1]
    h1 = w2.shape[0]
    oc = w2.shape[1]
    bd = min(nd, 512)
    grid = nd // bd
    pst = pos_src.T.reshape(3, ns)

    in_specs = [
        pl.BlockSpec((bd, 3), lambda i: (i, 0)),
        pl.BlockSpec((3, ns), lambda i: (0, 0)),
        pl.BlockSpec((ns, c), lambda i: (0, 0)),
        pl.BlockSpec((bd, s), lambda i: (i, 0)),
        pl.BlockSpec((c, h1), lambda i: (0, 0)),
        pl.BlockSpec((s, h1), lambda i: (0, 0)),
        pl.BlockSpec((1, h1), lambda i: (0, 0)),
        pl.BlockSpec((h1, oc), lambda i: (0, 0)),
        pl.BlockSpec((1, oc), lambda i: (0, 0)),
    ]
    args = [pos_dst, pst, x_src, skip, wi, ws, b1.reshape(1, h1), w2,
            b2.reshape(1, oc)]
    out_c = oc
    if three_layers:
        (w3, b3) = params[2]
        out_c = w3.shape[1]
        in_specs += [
            pl.BlockSpec((oc, out_c), lambda i: (0, 0)),
            pl.BlockSpec((1, out_c), lambda i: (0, 0)),
        ]
        args += [w3, b3.reshape(1, out_c)]

    return pl.pallas_call(
        functools.partial(_fp_kernel, k=k, three_layers=three_layers),
        grid=(grid,),
        in_specs=in_specs,
        out_specs=pl.BlockSpec((bd, out_c), lambda i: (i, 0)),
        out_shape=jax.ShapeDtypeStruct((nd, out_c), F32),
    )(*args)


def kernel(x, pos, batch, params):
    del batch
    samp1 = _fps(pos, pos.shape[0] // 4)
    pos1 = jnp.take(pos, samp1, axis=0)
    x1 = _sa_module(x, pos, pos.T, pos1, params['sa1'], 0.2)

    samp2 = _fps(pos1, pos1.shape[0] // 4)
    pos2 = jnp.take(pos1, samp2, axis=0)
    x2 = _sa_module(x1, pos1, pos1.T, pos2, params['sa2'], 0.4)

    samp3 = _fps(pos2, pos2.shape[0] // 4)
    pos3 = jnp.take(pos2, samp3, axis=0)
    x3 = _sa_module(x2, pos2, pos2.T, pos3, params['sa3'], 0.8)

    f3 = _fp_module(x3, pos3, pos2, x2, params['fp3'], 1)
    f2 = _fp_module(f3, pos2, pos1, x1, params['fp2'], 3)
    f1 = _fp_module(f2, pos1, pos, x, params['fp1'], 3)
    return f1
